# baseline (speedup 1.0000x reference)
import jax
import jax.numpy as jnp
from jax.experimental import pallas as pl

T, D, S = 16384, 2048, 16
T_TILE = 256
NSTREAM = 8
NCHUNK = T // T_TILE


def _tc_body(*refs):
    x_refs = refs[:NSTREAM]
    g_refs = refs[NSTREAM:2 * NSTREAM]
    wat_ref, ws_ref, batt_ref, abias_ref, bstrat_ref, out_ref = refs[2 * NSTREAM:]
    wa = wat_ref[...].T
    wst = ws_ref[...].reshape(S, D).T
    wc = jnp.concatenate([wa, wst], axis=1)
    battT = (batt_ref[...] + abias_ref[...]).T
    bstratT = bstrat_ref[...].T
    for j in range(NSTREAM):
        acc = jnp.dot(x_refs[j][...], wc, preferred_element_type=jnp.float32)
        accT = acc.T
        z = accT[:S, :] + battT + g_refs[j][...]
        m = jnp.max(z, axis=0, keepdims=True)
        e = jnp.exp(z - m)
        den = jnp.sum(e, axis=0, keepdims=True)
        sig = accT[S:, :] + bstratT
        num = jnp.sum(e * sig, axis=0, keepdims=True)
        out_ref[j:j + 1, :] = num / den


@jax.jit
def kernel(x, gumbel_noise, W_att, b_att, W_strat, b_strat, adaptive_bias):
    wat = W_att.T
    ws = W_strat.reshape(S * D)
    gt = gumbel_noise.T
    batt = b_att.reshape(1, S)
    abias = adaptive_bias.reshape(1, S)
    bstrat = b_strat.reshape(1, S)
    grid = (T // (NSTREAM * T_TILE),)

    def xmap(j):
        return lambda i: (NSTREAM * i + j, 0)

    def gmap(j):
        return lambda i: (0, NSTREAM * i + j)

    out = pl.pallas_call(
        _tc_body,
        grid=grid,
        in_specs=(
            [pl.BlockSpec((T_TILE, D), xmap(j)) for j in range(NSTREAM)]
            + [pl.BlockSpec((S, T_TILE), gmap(j)) for j in range(NSTREAM)]
            + [
                pl.BlockSpec((S, D), lambda i: (0, 0)),
                pl.BlockSpec((S * D,), lambda i: (0,)),
                pl.BlockSpec((1, S), lambda i: (0, 0)),
                pl.BlockSpec((1, S), lambda i: (0, 0)),
                pl.BlockSpec((1, S), lambda i: (0, 0)),
            ]
        ),
        out_specs=pl.BlockSpec((NSTREAM, T_TILE), lambda i: (i, 0)),
        out_shape=jax.ShapeDtypeStruct((NCHUNK, T_TILE), jnp.float32),
    )(*([x] * NSTREAM + [gt] * NSTREAM + [wat, ws, batt, abias, bstrat]))
    return out.reshape(T, 1)
